# baseline (speedup 1.0000x reference)
import functools
import jax
import jax.numpy as jnp
from jax import lax
from jax.experimental import pallas as pl
from jax.experimental.pallas import tpu as pltpu
from jax.experimental.pallas import tpu_sc as plsc


def _make_sc_probe():
    mesh = plsc.VectorSubcoreMesh(core_axis_name="c", subcore_axis_name="s")

    @functools.partial(
        pl.kernel, mesh=mesh,
        out_type=jax.ShapeDtypeStruct((128, 8), jnp.float32),
        scratch_types=[pltpu.VMEM((128, 8), jnp.float32)],
    )
    def sc_copy(x_hbm, o_hbm, buf):
        wid = lax.axis_index("s") * 2 + lax.axis_index("c")

        @pl.when(wid == 0)
        def _():
            pltpu.sync_copy(x_hbm, buf)
            pltpu.sync_copy(buf, o_hbm)

    return sc_copy

_N = 1500
_M = 128
_D_MOTION = 13
_N_TYPES = 10
_N_ATTRS = 8
_CLIP_LO = 1e-7
_CLIP_HI = 1.0 - 1e-7
_BIG = 1 << 22
_HI = lax.Precision.HIGHEST


def _t(x, eye):
    return lax.dot_general(x, eye, (((0,), (0,)), ((), ())),
                           preferred_element_type=jnp.float32, precision=_HI)


def _loss_body(pm_ref, plog_ref, pa_ref, gm_ref, ga_ref, gty_ref,
               o0_ref, o1_ref, o2_ref, o3_ref, o4_ref, cost_ref):
    f32 = jnp.float32
    i32 = jnp.int32
    inf = f32(jnp.inf)
    lane = lax.broadcasted_iota(i32, (1, _M), 1)
    rowm = lax.broadcasted_iota(i32, (_M, 1), 0)
    rown = lax.broadcasted_iota(i32, (_N, 1), 0)
    rownf = rown.astype(f32)
    eye = (rowm == lane).astype(f32)

    gmt = _t(gm_ref[...], eye)
    gx = gmt[0:1, :]
    gy = gmt[1:2, :]
    gz = gmt[2:3, :]

    d0 = pm_ref[:, 0:1] - gx
    d1 = pm_ref[:, 1:2] - gy
    d2 = pm_ref[:, 2:3] - gz
    cost_ref[...] = (d0 * d0 + d1 * d1) + d2 * d2

    picks0 = jnp.zeros((1, _M), i32)
    k0 = jnp.zeros((1, 1), i32)

    def cond(carry):
        k, _ = carry
        return k[0, 0] < _M

    def round_body(carry):
        k, picks = carry
        c = cost_ref[...]
        mn = jnp.min(c, axis=0, keepdims=True)
        pv = jnp.where(c <= mn, rown, i32(_BIG))
        p = jnp.min(pv, axis=0, keepdims=True)
        active = lane >= k
        activet = rowm >= k
        claims = (pv == p) & active
        claimsf = claims.astype(f32)
        pt = lax.dot_general(claimsf, rownf, (((0,), (0,)), ((), ())),
                             preferred_element_type=f32, precision=_HI)
        eq = (pt == p.astype(f32)) & activet & (rowm < lane)
        conf = jnp.any(eq, axis=0, keepdims=True) & active
        newk = jnp.min(jnp.where(conf, lane, i32(_M)), axis=(0, 1),
                       keepdims=True)
        newly = active & (lane < newk)
        newlytf = (activet & (rowm < newk)).astype(f32)
        picks = jnp.where(newly, p, picks)
        marks = jnp.dot(claimsf, newlytf, preferred_element_type=f32)
        cost_ref[...] = jnp.where(marks > 0.0, inf, c)
        return newk, picks

    _, picks = lax.while_loop(cond, round_body, (k0, picks0))

    oh = (rown == picks).astype(f32)
    pfcat = jnp.concatenate([pm_ref[...], plog_ref[...], pa_ref[...]],
                            axis=1)
    featst = lax.dot_general(pfcat, oh, (((0,), (0,)), ((), ())),
                             preferred_element_type=f32, precision=_HI)
    mmt = featst[0:13, :]
    mlt = featst[13:23, :]
    mat = featst[23:31, :]

    dmm = mmt - gmt
    motion_loss = jnp.sum(dmm * dmm) / f32(_M * _D_MOTION)

    mx = jnp.max(mlt, axis=0, keepdims=True)
    lse = mx + jnp.log(jnp.sum(jnp.exp(mlt - mx), axis=0, keepdims=True))
    toht = (lax.broadcasted_iota(i32, (_N_TYPES, _M), 0)
            == gty_ref[...].reshape(1, _M)).astype(f32)
    type_loss = (jnp.sum(lse) - jnp.sum(mlt * toht)) / f32(_M)

    gat = _t(ga_ref[...], eye)
    mac = jnp.clip(mat, _CLIP_LO, _CLIP_HI)
    bce = -(gat * jnp.log(mac) + (1.0 - gat) * jnp.log(1.0 - mac))
    attr_loss = jnp.sum(bce) / f32(_M * _N_ATTRS)

    alane = lax.broadcasted_iota(i32, (_N, _N_ATTRS), 1)
    pac = jnp.clip(pa_ref[...], _CLIP_LO, _CLIP_HI)
    base = jnp.sum(jnp.where(alane == 0, -jnp.log(1.0 - pac), f32(0.0)))
    pem = jnp.clip(mat[0:1, :], _CLIP_LO, _CLIP_HI)
    corr = jnp.sum(jnp.log(1.0 - pem) - jnp.log(pem))
    exist_loss = (base + corr) / f32(_N)

    total = (motion_loss + 0.5 * type_loss + 0.5 * attr_loss
             + 2.0 * exist_loss)
    o0_ref[0] = total
    o1_ref[0] = motion_loss
    o2_ref[0] = type_loss
    o3_ref[0] = attr_loss
    o4_ref[0] = exist_loss


def kernel(pred_motion, pred_type_logits, pred_attributes, gt_motion,
           gt_attributes, gt_type):
    f32 = jnp.float32

    out = pl.pallas_call(
        _loss_body,
        out_shape=[jax.ShapeDtypeStruct((1,), f32)] * 5,
        in_specs=[
            pl.BlockSpec(memory_space=pltpu.VMEM),
            pl.BlockSpec(memory_space=pltpu.VMEM),
            pl.BlockSpec(memory_space=pltpu.VMEM),
            pl.BlockSpec(memory_space=pltpu.VMEM),
            pl.BlockSpec(memory_space=pltpu.VMEM),
            pl.BlockSpec(memory_space=pltpu.VMEM),
        ],
        out_specs=[pl.BlockSpec(memory_space=pltpu.SMEM)] * 5,
        scratch_shapes=[
            pltpu.VMEM((_N, _M), f32),
        ],
    )(pred_motion.astype(f32), pred_type_logits.astype(f32),
      pred_attributes.astype(f32), gt_motion.astype(f32),
      gt_attributes.astype(f32), gt_type.astype(jnp.int32))

    scy = _make_sc_probe()(gt_attributes.astype(f32))
    return (out[0][0] + 0.0 * scy[0, 0], out[1][0], out[2][0], out[3][0],
            out[4][0])


# speedup vs baseline: 2.2685x; 2.2685x over previous
import jax
import jax.numpy as jnp
from jax import lax
from jax.experimental import pallas as pl
from jax.experimental.pallas import tpu as pltpu

_N = 1500
_M = 128
_D_MOTION = 13
_N_TYPES = 10
_N_ATTRS = 8
_CLIP_LO = 1e-7
_CLIP_HI = 1.0 - 1e-7
_BIG = 1 << 22
_HI = lax.Precision.HIGHEST


def _t(x, eye):
    return lax.dot_general(x, eye, (((0,), (0,)), ((), ())),
                           preferred_element_type=jnp.float32, precision=_HI)


def _loss_body(pm_ref, plog_ref, pa_ref, gm_ref, ga_ref, gty_ref,
               o0_ref, o1_ref, o2_ref, o3_ref, o4_ref, cost_ref):
    f32 = jnp.float32
    i32 = jnp.int32
    inf = f32(jnp.inf)
    lane = lax.broadcasted_iota(i32, (1, _M), 1)
    rowm = lax.broadcasted_iota(i32, (_M, 1), 0)
    rown = lax.broadcasted_iota(i32, (_N, 1), 0)
    rlo = (rown & 255).astype(f32)
    rhi = (rown >> 8).astype(f32)
    eye = (rowm == lane).astype(f32)

    gmt = _t(gm_ref[...], eye)
    gx = gmt[0:1, :]
    gy = gmt[1:2, :]
    gz = gmt[2:3, :]

    d0 = pm_ref[:, 0:1] - gx
    d1 = pm_ref[:, 1:2] - gy
    d2 = pm_ref[:, 2:3] - gz
    cost_ref[...] = (d0 * d0 + d1 * d1) + d2 * d2

    picks0 = jnp.zeros((1, _M), i32)
    k0 = jnp.zeros((1, 1), i32)

    def cond(carry):
        k, _ = carry
        return k[0, 0] < _M

    def round_body(carry):
        k, picks = carry
        c = cost_ref[...]
        mn = jnp.min(c, axis=0, keepdims=True)
        pv = jnp.where(c <= mn, rown, i32(_BIG))
        p = jnp.min(pv, axis=0, keepdims=True)
        active = lane >= k
        activet = rowm >= k
        claims = (pv == p) & active
        claimsf = claims.astype(f32)
        ptlo = lax.dot_general(claimsf, rlo, (((0,), (0,)), ((), ())),
                               preferred_element_type=f32)
        pthi = lax.dot_general(claimsf, rhi, (((0,), (0,)), ((), ())),
                               preferred_element_type=f32)
        pt = pthi * 256.0 + ptlo
        eq = (pt == p.astype(f32)) & activet & (rowm < lane)
        conf = jnp.any(eq, axis=0, keepdims=True) & active
        newk = jnp.min(jnp.where(conf, lane, i32(_M)), axis=(0, 1),
                       keepdims=True)
        newly = active & (lane < newk)
        newlytf = (activet & (rowm < newk)).astype(f32)
        picks = jnp.where(newly, p, picks)
        marks = jnp.dot(claimsf, newlytf, preferred_element_type=f32)
        cost_ref[...] = jnp.where(marks > 0.0, inf, c)
        return newk, picks

    _, picks = lax.while_loop(cond, round_body, (k0, picks0))

    oh = (rown == picks).astype(f32)
    pfcat = jnp.concatenate([pm_ref[...], plog_ref[...], pa_ref[...]],
                            axis=1)
    featst = lax.dot_general(pfcat, oh, (((0,), (0,)), ((), ())),
                             preferred_element_type=f32, precision=_HI)
    mmt = featst[0:13, :]
    mlt = featst[13:23, :]
    mat = featst[23:31, :]

    dmm = mmt - gmt
    motion_loss = jnp.sum(dmm * dmm) / f32(_M * _D_MOTION)

    mx = jnp.max(mlt, axis=0, keepdims=True)
    lse = mx + jnp.log(jnp.sum(jnp.exp(mlt - mx), axis=0, keepdims=True))
    toht = (lax.broadcasted_iota(i32, (_N_TYPES, _M), 0)
            == gty_ref[...].reshape(1, _M)).astype(f32)
    type_loss = (jnp.sum(lse) - jnp.sum(mlt * toht)) / f32(_M)

    gat = _t(ga_ref[...], eye)
    mac = jnp.clip(mat, _CLIP_LO, _CLIP_HI)
    bce = -(gat * jnp.log(mac) + (1.0 - gat) * jnp.log(1.0 - mac))
    attr_loss = jnp.sum(bce) / f32(_M * _N_ATTRS)

    alane = lax.broadcasted_iota(i32, (_N, _N_ATTRS), 1)
    pac = jnp.clip(pa_ref[...], _CLIP_LO, _CLIP_HI)
    base = jnp.sum(jnp.where(alane == 0, -jnp.log(1.0 - pac), f32(0.0)))
    pem = jnp.clip(mat[0:1, :], _CLIP_LO, _CLIP_HI)
    corr = jnp.sum(jnp.log(1.0 - pem) - jnp.log(pem))
    exist_loss = (base + corr) / f32(_N)

    total = (motion_loss + 0.5 * type_loss + 0.5 * attr_loss
             + 2.0 * exist_loss)
    o0_ref[0] = total
    o1_ref[0] = motion_loss
    o2_ref[0] = type_loss
    o3_ref[0] = attr_loss
    o4_ref[0] = exist_loss


def kernel(pred_motion, pred_type_logits, pred_attributes, gt_motion,
           gt_attributes, gt_type):
    f32 = jnp.float32

    out = pl.pallas_call(
        _loss_body,
        out_shape=[jax.ShapeDtypeStruct((1,), f32)] * 5,
        in_specs=[
            pl.BlockSpec(memory_space=pltpu.VMEM),
            pl.BlockSpec(memory_space=pltpu.VMEM),
            pl.BlockSpec(memory_space=pltpu.VMEM),
            pl.BlockSpec(memory_space=pltpu.VMEM),
            pl.BlockSpec(memory_space=pltpu.VMEM),
            pl.BlockSpec(memory_space=pltpu.VMEM),
        ],
        out_specs=[pl.BlockSpec(memory_space=pltpu.SMEM)] * 5,
        scratch_shapes=[
            pltpu.VMEM((_N, _M), f32),
        ],
    )(pred_motion.astype(f32), pred_type_logits.astype(f32),
      pred_attributes.astype(f32), gt_motion.astype(f32),
      gt_attributes.astype(f32), gt_type.astype(jnp.int32))

    return (out[0][0], out[1][0], out[2][0], out[3][0], out[4][0])
